# baseline (speedup 1.0000x reference)
import jax
import jax.numpy as jnp
from jax import lax
from jax.experimental import pallas as pl
from jax.experimental.pallas import tpu as pltpu
from jax.experimental.pallas import tpu_sc as plsc

RADIUS = 2.5
H = 16
C = 256
CH = C // 2
G = 8
CPG = C // G
N = 10000

NC = 2
NS = 16
NW = NC * NS

MC = 2000
KC = N // MC

ROWSC = MC * H
RPW = ROWSC // NW
CHUNK = 128
NFULL = RPW // CHUNK
TAIL = RPW - NFULL * CHUNK
IDXC = RPW + 8
PW = 8

RPWG = N * H // NW
IDXCG = RPWG + 8

BM = 400
NBLK = MC // BM


def _pack_bf16(lo, hi):
    ul = lax.bitcast_convert_type(lo, jnp.uint32)
    uh = lax.bitcast_convert_type(hi, jnp.uint32)
    ul = (ul + jnp.uint32(0x7FFF) + ((ul >> 16) & jnp.uint32(1))) >> 16
    uh = ((uh + jnp.uint32(0x7FFF) + ((uh >> 16) & jnp.uint32(1))) >> 16) << 16
    return lax.bitcast_convert_type(ul | uh, jnp.float32)


def _unpack_bf16(x):
    xi = lax.bitcast_convert_type(x, jnp.uint32)
    lo = lax.bitcast_convert_type(xi << 16, jnp.float32)
    hi = lax.bitcast_convert_type(xi & jnp.uint32(0xFFFF0000), jnp.float32)
    return lo, hi


def _fold_body(sf_ref, wg_ref, wd2_ref, bd2_ref, bg_ref,
               sfw_ref, w2g_ref, b2_ref):
    i = pl.program_id(0)
    r = jnp.dot(sf_ref[...], wg_ref[...], preferred_element_type=jnp.float32)
    sfw_ref[...] = _pack_bf16(r[:, :CH], r[:, CH:])

    @pl.when(i == 0)
    def _():
        w2g_ref[...] = jnp.dot(wd2_ref[...], wg_ref[...],
                               preferred_element_type=jnp.float32)
        b2_ref[...] = jnp.dot(bd2_ref[...], wg_ref[...],
                              preferred_element_type=jnp.float32) + bg_ref[...]


def _sc_pts_body(xs_hbm, ys_hbm, zs_hbm, idx_hbm, npts_hbm,
                 idx_v, xs_v, ys_v, zs_v, pts_v):
    wid = lax.axis_index("s") * NC + lax.axis_index("c")
    base = pl.multiple_of(wid * RPWG, 8)
    pltpu.sync_copy(idx_hbm.at[pl.ds(base, IDXCG)], idx_v)
    pltpu.sync_copy(xs_hbm, xs_v)
    pltpu.sync_copy(ys_hbm, ys_v)
    pltpu.sync_copy(zs_hbm, zs_v)
    lanes = jnp.arange(16, dtype=jnp.int32)

    def body(g, _):
        goff = pl.multiple_of(g * 16, 8)
        idx16 = idx_v[pl.ds(goff, 16)]
        lrow = lanes + goff
        m = lrow < RPWG
        x = plsc.load_gather(xs_v, [idx16])
        y = plsc.load_gather(ys_v, [idx16])
        z = plsc.load_gather(zs_v, [idx16])
        plsc.store_scatter(pts_v, [lrow * PW + 0], x, mask=m)
        plsc.store_scatter(pts_v, [lrow * PW + 1], y, mask=m)
        plsc.store_scatter(pts_v, [lrow * PW + 2], z, mask=m)
        return 0

    lax.fori_loop(0, (RPWG + 15) // 16, body, 0)
    pltpu.sync_copy(pts_v,
                    npts_hbm.at[pl.ds(pl.multiple_of(base * PW, 8),
                                      RPWG * PW)])


def _sc_rows_body(sfw_hbm, idx_hbm, nfw_hbm, idx_v, rows_v, sem0, sem1):
    wid = lax.axis_index("s") * NC + lax.axis_index("c")
    base = pl.multiple_of(wid * RPW, 8)
    pltpu.sync_copy(idx_hbm.at[pl.ds(base, IDXC)], idx_v)

    offs = [j * CHUNK for j in range(NFULL)] + [NFULL * CHUNK]
    sizes = [CHUNK] * NFULL + [TAIL]
    sems = [sem0, sem1]

    def start(j):
        return pltpu.async_copy(
            sfw_hbm.at[idx_v.at[pl.ds(offs[j], sizes[j])]],
            rows_v.at[j % 2].at[pl.ds(0, sizes[j])], sems[j % 2])

    cp = start(0)
    for j in range(len(offs)):
        nxt = start(j + 1) if j + 1 < len(offs) else None
        cp.wait()
        pltpu.sync_copy(rows_v.at[j % 2].at[pl.ds(0, sizes[j])],
                        nfw_hbm.at[pl.ds(base + offs[j], sizes[j])])
        cp = nxt


def _main_body(nfw_ref, npts_ref, q_ref, w1b_ref, bd1_ref, w2g_ref, b2_ref,
               wa1_ref, ba1_ref, wa2_ref, ba2_ref, out_ref):
    scale = 1.0 / RADIUS
    v = npts_ref[...]
    q = q_ref[...]

    tp_r = lax.broadcasted_iota(jnp.int32, (PW, H * PW), 0)
    tp_c = lax.broadcasted_iota(jnp.int32, (PW, H * PW), 1) % PW
    tpat = jnp.where(tp_r == tp_c, 1.0, 0.0).astype(jnp.float32)
    qrep = jnp.dot(q, tpat, preferred_element_type=jnp.float32)

    cmask = (lax.broadcasted_iota(jnp.int32, (BM, H * PW), 1) % PW) < 3
    nb = jnp.where(cmask, (v - qrep) * scale, 0.0)

    gp = jnp.dot(nb, w1b_ref[...], preferred_element_type=jnp.float32)
    gp = gp + bd1_ref[...]
    gp = jnp.where(gp >= 0, gp, 0.1 * gp)

    w2gb = w2g_ref[...].astype(jnp.bfloat16)
    b2 = b2_ref[...]
    nf = []
    for h in range(H):
        geh = jnp.dot(gp[:, h * C:(h + 1) * C].astype(jnp.bfloat16), w2gb,
                      preferred_element_type=jnp.float32)
        lo, hi = _unpack_bf16(nfw_ref[:, h * CH:(h + 1) * CH])
        nf.append(jnp.concatenate([lo, hi], axis=-1) + geh + b2)

    pooled = nf[0]
    a = jnp.dot(pooled, wa1_ref[...], preferred_element_type=jnp.float32)
    a = a + ba1_ref[...]
    a = jnp.where(a >= 0, a, 0.1 * a)
    aw = jnp.dot(a, wa2_ref[...], preferred_element_type=jnp.float32)
    aw = aw + ba2_ref[...]

    cols = lax.broadcasted_iota(jnp.int32, (CPG, C), 1) // G
    rows = lax.broadcasted_iota(jnp.int32, (CPG, C), 0)
    exp8 = jnp.where(cols == rows, 1.0, 0.0).astype(jnp.float32)

    acc = jnp.zeros((BM, C), jnp.float32)
    for h in range(H):
        aw_h = aw[:, h * CPG:(h + 1) * CPG]
        awx = jnp.dot(aw_h, exp8, preferred_element_type=jnp.float32)
        acc = acc + nf[h] * awx
    out_ref[...] = acc


def _stage_c(k, nfw, npts, q_pad, w1b, bd1r, w2g, b2, Wa1, ba1_2, Wa2,
             ba2_2):
    return pl.pallas_call(
        _main_body,
        grid=(NBLK,),
        in_specs=[
            pl.BlockSpec((BM, H * CH), lambda i: (i, 0)),
            pl.BlockSpec((BM, H * PW), lambda i, k=k: (k * NBLK + i, 0)),
            pl.BlockSpec((BM, PW), lambda i, k=k: (k * NBLK + i, 0)),
            pl.BlockSpec((H * PW, H * C), lambda i: (0, 0)),
            pl.BlockSpec((1, H * C), lambda i: (0, 0)),
            pl.BlockSpec((C, C), lambda i: (0, 0)),
            pl.BlockSpec((1, C), lambda i: (0, 0)),
            pl.BlockSpec((C, C), lambda i: (0, 0)),
            pl.BlockSpec((1, C), lambda i: (0, 0)),
            pl.BlockSpec((C, H * CPG), lambda i: (0, 0)),
            pl.BlockSpec((1, H * CPG), lambda i: (0, 0)),
        ],
        out_specs=pl.BlockSpec((BM, C), lambda i: (i, 0)),
        out_shape=jax.ShapeDtypeStruct((MC, C), jnp.float32),
    )(nfw, npts, q_pad, w1b, bd1r, w2g, b2, Wa1, ba1_2, Wa2, ba2_2)


@jax.jit
def kernel(q_pts, s_pts, s_feats, neighb_inds, Wd1, bd1, Wd2, bd2, Wg, bg,
           Wa1, ba1, Wa2, ba2):
    idx = neighb_inds.astype(jnp.int32)
    idx_flat = jnp.pad(idx.reshape(-1), (0, 64))
    xs = s_pts[:, 0]
    ys = s_pts[:, 1]
    zs = s_pts[:, 2]
    q_pts_pad = jnp.pad(q_pts, ((0, 0), (0, PW - 3)))
    wd1_pad = jnp.pad(Wd1, ((0, PW - 3), (0, 0)))
    w1b = jnp.kron(jnp.eye(H, dtype=jnp.float32), wd1_pad)
    bd1r = jnp.tile(bd1.reshape(1, C), (1, H))
    bd2_2 = bd2.reshape(1, C)
    bg_2 = bg.reshape(1, C)
    ba1_2 = ba1.reshape(1, C)
    ba2_2 = ba2.reshape(1, H * CPG)

    ab = 2000
    sfw, w2g, b2 = pl.pallas_call(
        _fold_body,
        grid=(N // ab,),
        in_specs=[
            pl.BlockSpec((ab, C), lambda i: (i, 0)),
            pl.BlockSpec((C, C), lambda i: (0, 0)),
            pl.BlockSpec((C, C), lambda i: (0, 0)),
            pl.BlockSpec((1, C), lambda i: (0, 0)),
            pl.BlockSpec((1, C), lambda i: (0, 0)),
        ],
        out_specs=[
            pl.BlockSpec((ab, CH), lambda i: (i, 0)),
            pl.BlockSpec((C, C), lambda i: (0, 0)),
            pl.BlockSpec((1, C), lambda i: (0, 0)),
        ],
        out_shape=[
            jax.ShapeDtypeStruct((N, CH), jnp.float32),
            jax.ShapeDtypeStruct((C, C), jnp.float32),
            jax.ShapeDtypeStruct((1, C), jnp.float32),
        ],
    )(s_feats, Wg, Wd2, bd2_2, bg_2)

    mesh = plsc.VectorSubcoreMesh(core_axis_name="c", subcore_axis_name="s")
    pts_fn = pl.kernel(
        _sc_pts_body,
        out_type=jax.ShapeDtypeStruct((N * H * PW,), jnp.float32),
        mesh=mesh,
        scratch_types=[
            pltpu.VMEM((IDXCG,), jnp.int32),
            pltpu.VMEM((N,), jnp.float32),
            pltpu.VMEM((N,), jnp.float32),
            pltpu.VMEM((N,), jnp.float32),
            pltpu.VMEM((RPWG * PW,), jnp.float32),
        ],
        compiler_params=pltpu.CompilerParams(needs_layout_passes=False),
    )
    rows_fn = pl.kernel(
        _sc_rows_body,
        out_type=jax.ShapeDtypeStruct((ROWSC, CH), jnp.float32),
        mesh=mesh,
        scratch_types=[
            pltpu.VMEM((IDXC,), jnp.int32),
            pltpu.VMEM((2, CHUNK, CH), jnp.float32),
            pltpu.SemaphoreType.DMA,
            pltpu.SemaphoreType.DMA,
        ],
        compiler_params=pltpu.CompilerParams(needs_layout_passes=False),
    )

    npts = pts_fn(xs, ys, zs, idx_flat).reshape(N, H * PW)

    nfws = []
    for k in range(KC):
        idx_k = idx_flat[k * ROWSC:(k + 1) * ROWSC + 64]
        nfws.append(rows_fn(sfw, idx_k).reshape(MC, H * CH))
    outs = []
    for k in range(KC):
        outs.append(_stage_c(k, nfws[k], npts, q_pts_pad, w1b, bd1r, w2g,
                             b2, Wa1, ba1_2, Wa2, ba2_2))
    return jnp.concatenate(outs, axis=0)


# speedup vs baseline: 1.1311x; 1.1311x over previous
import jax
import jax.numpy as jnp
from jax import lax
from jax.experimental import pallas as pl
from jax.experimental.pallas import tpu as pltpu
from jax.experimental.pallas import tpu_sc as plsc

RADIUS = 2.5
H = 16
C = 256
G = 8
CPG = C // G
N = 10000

NC = 2
NS = 16
NW = NC * NS

MC = 2000
KC = N // MC

ROWSC = MC * H
RPW = ROWSC // NW
CHUNK = 128
NFULL = RPW // CHUNK
TAIL = RPW - NFULL * CHUNK
IDXC = RPW + 8
PW = 8

RPWG = N * H // NW
IDXCG = RPWG + 8

BM = 400
NBLK = MC // BM


def _pack_bf16(lo, hi):
    ul = lax.bitcast_convert_type(lo, jnp.uint32)
    uh = lax.bitcast_convert_type(hi, jnp.uint32)
    ul = (ul + jnp.uint32(0x7FFF) + ((ul >> 16) & jnp.uint32(1))) >> 16
    uh = ((uh + jnp.uint32(0x7FFF) + ((uh >> 16) & jnp.uint32(1))) >> 16) << 16
    return lax.bitcast_convert_type(ul | uh, jnp.float32)


def _unpack_bf16(x):
    xi = lax.bitcast_convert_type(x, jnp.uint32)
    lo = lax.bitcast_convert_type(xi << 16, jnp.float32)
    hi = lax.bitcast_convert_type(xi & jnp.uint32(0xFFFF0000), jnp.float32)
    return lo, hi


def _fold_body(sf_ref, wg_ref, wd2_ref, bd2_ref, bg_ref,
               sfw_ref, w2g_ref, b2_ref):
    i = pl.program_id(0)
    r = jnp.dot(sf_ref[...], wg_ref[...], preferred_element_type=jnp.float32)
    sfw_ref[...] = _pack_bf16(r[:, :C // 2], r[:, C // 2:])

    @pl.when(i == 0)
    def _():
        w2g_ref[...] = jnp.dot(wd2_ref[...], wg_ref[...],
                               preferred_element_type=jnp.float32)
        b2_ref[...] = jnp.dot(bd2_ref[...], wg_ref[...],
                              preferred_element_type=jnp.float32) + bg_ref[...]


def _sc_pts_body(xs_hbm, ys_hbm, zs_hbm, idx_hbm, npts_hbm,
                 idx_v, xs_v, ys_v, zs_v, pts_v):
    wid = lax.axis_index("s") * NC + lax.axis_index("c")
    base = pl.multiple_of(wid * RPWG, 8)
    pltpu.sync_copy(idx_hbm.at[pl.ds(base, IDXCG)], idx_v)
    pltpu.sync_copy(xs_hbm, xs_v)
    pltpu.sync_copy(ys_hbm, ys_v)
    pltpu.sync_copy(zs_hbm, zs_v)
    lanes = jnp.arange(16, dtype=jnp.int32)

    def body(g, _):
        goff = pl.multiple_of(g * 16, 8)
        idx16 = idx_v[pl.ds(goff, 16)]
        lrow = lanes + goff
        m = lrow < RPWG
        x = plsc.load_gather(xs_v, [idx16])
        y = plsc.load_gather(ys_v, [idx16])
        z = plsc.load_gather(zs_v, [idx16])
        plsc.store_scatter(pts_v, [lrow * PW + 0], x, mask=m)
        plsc.store_scatter(pts_v, [lrow * PW + 1], y, mask=m)
        plsc.store_scatter(pts_v, [lrow * PW + 2], z, mask=m)
        return 0

    lax.fori_loop(0, (RPWG + 15) // 16, body, 0)
    pltpu.sync_copy(pts_v,
                    npts_hbm.at[pl.ds(pl.multiple_of(base * PW, 8),
                                      RPWG * PW)])


def _sc_rows_body(sfw_hbm, idx_hbm, nfw_hbm, idx_v, rows_v, sem0, sem1):
    wid = lax.axis_index("s") * NC + lax.axis_index("c")
    base = pl.multiple_of(wid * RPW, 8)
    pltpu.sync_copy(idx_hbm.at[pl.ds(base, IDXC)], idx_v)

    offs = [j * CHUNK for j in range(NFULL)] + [NFULL * CHUNK]
    sizes = [CHUNK] * NFULL + [TAIL]
    sems = [sem0, sem1]

    def start(j):
        return pltpu.async_copy(
            sfw_hbm.at[idx_v.at[pl.ds(offs[j], sizes[j])]],
            rows_v.at[j % 2].at[pl.ds(0, sizes[j])], sems[j % 2])

    cp = start(0)
    for j in range(len(offs)):
        nxt = start(j + 1) if j + 1 < len(offs) else None
        cp.wait()
        pltpu.sync_copy(rows_v.at[j % 2].at[pl.ds(0, sizes[j])],
                        nfw_hbm.at[pl.ds(base + offs[j], sizes[j])])
        cp = nxt


def _main_body(nfw_ref, npts_ref, q_ref, wd1_ref, bd1_ref, w2g_ref, b2_ref,
               wa1_ref, ba1_ref, wa2_ref, ba2_ref, out_ref):
    scale = 1.0 / RADIUS
    q = q_ref[...]
    wd1 = wd1_ref[...]
    bd1 = bd1_ref[...]
    w2g = w2g_ref[...]
    b2 = b2_ref[...]
    colmask = lax.broadcasted_iota(jnp.int32, (H * BM, PW), 1) < 3

    npts = npts_ref[...].reshape(H * BM, PW)
    nb = jnp.where(colmask, (npts - jnp.broadcast_to(q, (H, BM, PW))
                             .reshape(H * BM, PW)) * scale, 0.0)
    g = jnp.dot(nb, wd1, preferred_element_type=jnp.float32) + bd1
    g = jnp.where(g >= 0, g, 0.1 * g)
    ge = jnp.dot(g.astype(jnp.bfloat16), w2g.astype(jnp.bfloat16),
                 preferred_element_type=jnp.float32)
    nf_lo, nf_hi = _unpack_bf16(nfw_ref[...].reshape(H * BM, C // 2))
    nfp = jnp.concatenate([nf_lo, nf_hi], axis=-1) + ge + b2

    pooled = nfp[0:BM]
    a = jnp.dot(pooled, wa1_ref[...], preferred_element_type=jnp.float32)
    a = a + ba1_ref[...]
    a = jnp.where(a >= 0, a, 0.1 * a)
    aw = jnp.dot(a, wa2_ref[...], preferred_element_type=jnp.float32)
    aw = aw + ba2_ref[...]

    cols = lax.broadcasted_iota(jnp.int32, (CPG, C), 1) // G
    rows = lax.broadcasted_iota(jnp.int32, (CPG, C), 0)
    exp8 = jnp.where(cols == rows, 1.0, 0.0).astype(jnp.float32)

    acc = jnp.zeros((BM, C), jnp.float32)
    for h in range(H):
        aw_h = aw[:, h * CPG:(h + 1) * CPG]
        awx = jnp.dot(aw_h, exp8, preferred_element_type=jnp.float32)
        acc = acc + nfp[h * BM:(h + 1) * BM] * awx
    out_ref[...] = acc


def _stage_c(k, nfw, npts, q_pad, wd1_pad, bd1_2, w2g, b2, Wa1, ba1_2, Wa2,
             ba2_2):
    return pl.pallas_call(
        _main_body,
        grid=(NBLK,),
        in_specs=[
            pl.BlockSpec((H, BM, C // 2), lambda i: (0, i, 0)),
            pl.BlockSpec((H, BM, PW), lambda i, k=k: (0, k * NBLK + i, 0)),
            pl.BlockSpec((BM, PW), lambda i, k=k: (k * NBLK + i, 0)),
            pl.BlockSpec((PW, C), lambda i: (0, 0)),
            pl.BlockSpec((1, C), lambda i: (0, 0)),
            pl.BlockSpec((C, C), lambda i: (0, 0)),
            pl.BlockSpec((1, C), lambda i: (0, 0)),
            pl.BlockSpec((C, C), lambda i: (0, 0)),
            pl.BlockSpec((1, C), lambda i: (0, 0)),
            pl.BlockSpec((C, H * CPG), lambda i: (0, 0)),
            pl.BlockSpec((1, H * CPG), lambda i: (0, 0)),
        ],
        out_specs=pl.BlockSpec((BM, C), lambda i: (i, 0)),
        out_shape=jax.ShapeDtypeStruct((MC, C), jnp.float32),
    )(nfw, npts, q_pad, wd1_pad, bd1_2, w2g, b2, Wa1, ba1_2, Wa2, ba2_2)


@jax.jit
def kernel(q_pts, s_pts, s_feats, neighb_inds, Wd1, bd1, Wd2, bd2, Wg, bg,
           Wa1, ba1, Wa2, ba2):
    idx = neighb_inds.astype(jnp.int32)
    xs = s_pts[:, 0]
    ys = s_pts[:, 1]
    zs = s_pts[:, 2]
    q_pts_pad = jnp.pad(q_pts, ((0, 0), (0, PW - 3)))
    wd1_pad = jnp.pad(Wd1, ((0, PW - 3), (0, 0)))
    bd1_2 = bd1.reshape(1, C)
    bd2_2 = bd2.reshape(1, C)
    bg_2 = bg.reshape(1, C)
    ba1_2 = ba1.reshape(1, C)
    ba2_2 = ba2.reshape(1, H * CPG)

    ab = 2000
    sfw, w2g, b2 = pl.pallas_call(
        _fold_body,
        grid=(N // ab,),
        in_specs=[
            pl.BlockSpec((ab, C), lambda i: (i, 0)),
            pl.BlockSpec((C, C), lambda i: (0, 0)),
            pl.BlockSpec((C, C), lambda i: (0, 0)),
            pl.BlockSpec((1, C), lambda i: (0, 0)),
            pl.BlockSpec((1, C), lambda i: (0, 0)),
        ],
        out_specs=[
            pl.BlockSpec((ab, C // 2), lambda i: (i, 0)),
            pl.BlockSpec((C, C), lambda i: (0, 0)),
            pl.BlockSpec((1, C), lambda i: (0, 0)),
        ],
        out_shape=[
            jax.ShapeDtypeStruct((N, C // 2), jnp.float32),
            jax.ShapeDtypeStruct((C, C), jnp.float32),
            jax.ShapeDtypeStruct((1, C), jnp.float32),
        ],
    )(s_feats, Wg, Wd2, bd2_2, bg_2)

    mesh = plsc.VectorSubcoreMesh(core_axis_name="c", subcore_axis_name="s")
    pts_fn = pl.kernel(
        _sc_pts_body,
        out_type=jax.ShapeDtypeStruct((N * H * PW,), jnp.float32),
        mesh=mesh,
        scratch_types=[
            pltpu.VMEM((IDXCG,), jnp.int32),
            pltpu.VMEM((N,), jnp.float32),
            pltpu.VMEM((N,), jnp.float32),
            pltpu.VMEM((N,), jnp.float32),
            pltpu.VMEM((RPWG * PW,), jnp.float32),
        ],
        compiler_params=pltpu.CompilerParams(needs_layout_passes=False),
    )
    rows_fn = pl.kernel(
        _sc_rows_body,
        out_type=jax.ShapeDtypeStruct((ROWSC, C // 2), jnp.float32),
        mesh=mesh,
        scratch_types=[
            pltpu.VMEM((IDXC,), jnp.int32),
            pltpu.VMEM((2, CHUNK, C // 2), jnp.float32),
            pltpu.SemaphoreType.DMA,
            pltpu.SemaphoreType.DMA,
        ],
        compiler_params=pltpu.CompilerParams(needs_layout_passes=False),
    )

    idx_g = jnp.pad(idx.T.reshape(-1), (0, 64))
    npts = pts_fn(xs, ys, zs, idx_g).reshape(H, N, PW)

    nfws = []
    for k in range(KC):
        idx_k = idx[k * MC:(k + 1) * MC, :].T.reshape(-1)
        idx_k = jnp.pad(idx_k, (0, 64))
        nfws.append(rows_fn(sfw, idx_k).reshape(H, MC, C // 2))
    outs = []
    for k in range(KC):
        outs.append(_stage_c(k, nfws[k], npts, q_pts_pad, wd1_pad, bd1_2,
                             w2g, b2, Wa1, ba1_2, Wa2, ba2_2))
    return jnp.concatenate(outs, axis=0)
